# baseline (device time: 5273174 ns/iter reference)
import jax
import jax.numpy as jnp
from jax import lax
from jax.experimental import pallas as pl
from jax.experimental.pallas import tpu as pltpu

N_DEV = 8
HOPS = N_DEV - 1


def _ring_allgather(x_shard):
    m_per, k = x_shard.shape
    m_half = m_per // 2
    dt = x_shard.dtype

    def body(x_ref, out_ref, bufr_ref, bufl_ref,
             seed_sem, own_sem, cr_sem, cl_sem, sr, rr, sl, rl):
        my = lax.axis_index("i")
        right = lax.rem(my + 1, N_DEV)
        left = lax.rem(my - 1 + N_DEV, N_DEV)

        seed_r = pltpu.make_async_copy(
            x_ref.at[pl.ds(0, m_half), :], bufr_ref.at[0], seed_sem)
        seed_l = pltpu.make_async_copy(
            x_ref.at[pl.ds(m_half, m_half), :], bufl_ref.at[0], seed_sem)
        own = pltpu.make_async_copy(
            x_ref, out_ref.at[pl.ds(my * m_per, m_per), :], own_sem)
        seed_r.start()
        seed_l.start()
        own.start()
        seed_r.wait()
        seed_l.wait()

        for h in range(HOPS):
            s, d = h % 2, (h + 1) % 2
            if h >= 1:
                pltpu.make_async_copy(bufr_ref.at[d], bufr_ref.at[d], cr_sem).wait()
                pltpu.make_async_copy(bufl_ref.at[d], bufl_ref.at[d], cl_sem).wait()
            rdma_r = pltpu.make_async_remote_copy(
                src_ref=bufr_ref.at[s], dst_ref=bufr_ref.at[d],
                send_sem=sr.at[h], recv_sem=rr.at[h],
                device_id=(right,), device_id_type=pl.DeviceIdType.MESH)
            rdma_l = pltpu.make_async_remote_copy(
                src_ref=bufl_ref.at[s], dst_ref=bufl_ref.at[d],
                send_sem=sl.at[h], recv_sem=rl.at[h],
                device_id=(left,), device_id_type=pl.DeviceIdType.MESH)
            rdma_r.start()
            rdma_l.start()
            rdma_r.wait()
            rdma_l.wait()
            origin_r = lax.rem(my - h - 1 + N_DEV, N_DEV)
            origin_l = lax.rem(my + h + 1, N_DEV)
            pltpu.make_async_copy(
                bufr_ref.at[d],
                out_ref.at[pl.ds(origin_r * m_per, m_half), :], cr_sem).start()
            pltpu.make_async_copy(
                bufl_ref.at[d],
                out_ref.at[pl.ds(origin_l * m_per + m_half, m_half), :],
                cl_sem).start()

        d = HOPS % 2
        pltpu.make_async_copy(bufr_ref.at[d], bufr_ref.at[d], cr_sem).wait()
        pltpu.make_async_copy(bufl_ref.at[d], bufl_ref.at[d], cl_sem).wait()
        own.wait()

    out, _, _ = pl.pallas_call(
        body,
        out_shape=[
            jax.ShapeDtypeStruct((N_DEV * m_per, k), dt),
            jax.ShapeDtypeStruct((2, m_half, k), dt),
            jax.ShapeDtypeStruct((2, m_half, k), dt),
        ],
        in_specs=[pl.BlockSpec(memory_space=pl.ANY)],
        out_specs=[pl.BlockSpec(memory_space=pl.ANY)] * 3,
        scratch_shapes=[
            pltpu.SemaphoreType.DMA,
            pltpu.SemaphoreType.DMA,
            pltpu.SemaphoreType.DMA,
            pltpu.SemaphoreType.DMA,
            pltpu.SemaphoreType.DMA((HOPS,)),
            pltpu.SemaphoreType.DMA((HOPS,)),
            pltpu.SemaphoreType.DMA((HOPS,)),
            pltpu.SemaphoreType.DMA((HOPS,)),
        ],
    )(x_shard)
    return out


def kernel(x, w_mat):
    x_full = _ring_allgather(x.astype(jnp.bfloat16))
    y = jnp.dot(x_full, w_mat.astype(jnp.bfloat16),
                preferred_element_type=jnp.float32)
    return y * jax.nn.sigmoid(y)


# device time: 778107 ns/iter; 6.7769x vs baseline; 6.7769x over previous
import jax
import jax.numpy as jnp
from jax import lax
from jax.experimental import pallas as pl
from jax.experimental.pallas import tpu as pltpu

N_DEV = 8
HOPS = N_DEV - 1


def _ring_allgather_rel(x_shard):
    m_per, k = x_shard.shape
    m_half = m_per // 2

    def body(x_ref, out_ref, seed_sem, sr, rr, sl, rl):
        my = lax.axis_index("i")
        right = lax.rem(my + 1, N_DEV)
        left = lax.rem(my - 1 + N_DEV, N_DEV)

        barrier_sem = pltpu.get_barrier_semaphore()
        for nbr in (left, right):
            pl.semaphore_signal(
                barrier_sem, inc=1,
                device_id=(nbr,), device_id_type=pl.DeviceIdType.MESH)
        pl.semaphore_wait(barrier_sem, 2)

        seed = pltpu.make_async_copy(
            x_ref, out_ref.at[pl.ds(0, m_per), :], seed_sem)
        seed.start()
        seed.wait()

        for h in range(HOPS):
            rt_s = (N_DEV - h) % N_DEV
            rdma_r = pltpu.make_async_remote_copy(
                src_ref=out_ref.at[pl.ds(rt_s * m_per, m_half), :],
                dst_ref=out_ref.at[pl.ds((HOPS - h) * m_per, m_half), :],
                send_sem=sr.at[h], recv_sem=rr.at[h],
                device_id=(right,), device_id_type=pl.DeviceIdType.MESH)
            rdma_l = pltpu.make_async_remote_copy(
                src_ref=out_ref.at[pl.ds(h * m_per + m_half, m_half), :],
                dst_ref=out_ref.at[pl.ds((h + 1) * m_per + m_half, m_half), :],
                send_sem=sl.at[h], recv_sem=rl.at[h],
                device_id=(left,), device_id_type=pl.DeviceIdType.MESH)
            rdma_r.start()
            rdma_l.start()
            rdma_r.wait()
            rdma_l.wait()

    return pl.pallas_call(
        body,
        out_shape=jax.ShapeDtypeStruct((N_DEV * m_per, k), x_shard.dtype),
        in_specs=[pl.BlockSpec(memory_space=pl.ANY)],
        out_specs=pl.BlockSpec(memory_space=pl.ANY),
        scratch_shapes=[
            pltpu.SemaphoreType.DMA,
            pltpu.SemaphoreType.DMA((HOPS,)),
            pltpu.SemaphoreType.DMA((HOPS,)),
            pltpu.SemaphoreType.DMA((HOPS,)),
            pltpu.SemaphoreType.DMA((HOPS,)),
        ],
        compiler_params=pltpu.CompilerParams(collective_id=0),
    )(x_shard)


def kernel(x, w_mat):
    x_rel = _ring_allgather_rel(x.astype(jnp.bfloat16))
    y = jnp.dot(x_rel, w_mat.astype(jnp.bfloat16),
                preferred_element_type=jnp.float32)
    y = y * jax.nn.sigmoid(y)
    return jnp.roll(y, lax.axis_index("i") * x.shape[0], axis=0)
